# Initial kernel scaffold; baseline (speedup 1.0000x reference)
import jax
import jax.numpy as jnp
from jax.experimental import pallas as pl


def kernel(phi_a, phi_b):
    raise NotImplementedError("write your pallas kernel here")


# speedup vs baseline: 7.0516x; 7.0516x over previous
import jax
import jax.numpy as jnp
from jax import lax
from jax.experimental import pallas as pl

_L1, _L2 = 512, 512
_H = _L2 // 2


def _perm_mats():
    k = jnp.arange(_H)[:, None]
    j = jnp.arange(_L2)[None, :]
    p0 = (j == 2 * k).astype(jnp.float32)
    p1 = (j == 2 * k + 1).astype(jnp.float32)
    return p0, p1


def _body(a_ref, b_ref, p0_ref, p1_ref, o_ref):
    a = a_ref[0]
    b = b_ref[0]
    rp = (lax.broadcasted_iota(jnp.int32, (_L1, _H), 0) % 2) == 0
    c = jnp.where(rp, a, b)
    d = jnp.where(rp, b, a)
    o_ref[0] = (
        jnp.dot(c, p0_ref[...], precision=lax.Precision.HIGHEST)
        + jnp.dot(d, p1_ref[...], precision=lax.Precision.HIGHEST)
    )


def kernel(phi_a, phi_b):
    B = phi_a.shape[0]
    a3 = phi_a.reshape(B, _L1, _H)
    b3 = phi_b.reshape(B, _L1, _H)
    p0, p1 = _perm_mats()
    return pl.pallas_call(
        _body,
        grid=(B,),
        in_specs=[
            pl.BlockSpec((1, _L1, _H), lambda b: (b, 0, 0)),
            pl.BlockSpec((1, _L1, _H), lambda b: (b, 0, 0)),
            pl.BlockSpec((_H, _L2), lambda b: (0, 0)),
            pl.BlockSpec((_H, _L2), lambda b: (0, 0)),
        ],
        out_specs=pl.BlockSpec((1, _L1, _L2), lambda b: (b, 0, 0)),
        out_shape=jax.ShapeDtypeStruct((B, _L1, _L2), phi_a.dtype),
    )(a3, b3, p0, p1)


# speedup vs baseline: 10.3816x; 1.4722x over previous
import functools

import jax
import jax.numpy as jnp
from jax import lax
from jax.experimental import pallas as pl
from jax.experimental.pallas import tpu as pltpu
from jax.experimental.pallas import tpu_sc as plsc

_B = 128
_L1, _L2 = 512, 512
_H = _L2 // 2

_NC, _NS = 2, 16
_NW = _NC * _NS
_NB_W = _B // _NW
_NR = 32
_NCH = _L1 // _NR
_NT = _NB_W * _NCH

_mesh = plsc.VectorSubcoreMesh(core_axis_name="c", subcore_axis_name="s")


@functools.partial(
    pl.kernel,
    out_type=jax.ShapeDtypeStruct((_B, _L1, _L2), jnp.float32),
    mesh=_mesh,
    scratch_types=[
        pltpu.VMEM((_NR * _H,), jnp.float32),
        pltpu.VMEM((_NR * _H,), jnp.float32),
        pltpu.VMEM((_NR * _H,), jnp.float32),
        pltpu.VMEM((_NR * _H,), jnp.float32),
        pltpu.VMEM((_NR, _L2), jnp.float32),
        pltpu.VMEM((_NR, _L2), jnp.float32),
        pltpu.SemaphoreType.DMA,
        pltpu.SemaphoreType.DMA,
        pltpu.SemaphoreType.DMA,
        pltpu.SemaphoreType.DMA,
    ],
    compiler_params=pltpu.CompilerParams(needs_layout_passes=False),
)
def _sc_restore(a_hbm, b_hbm, out_hbm,
                a_v0, a_v1, b_v0, b_v1, o_v0, o_v1,
                si0, si1, so0, so1):
    cid = lax.axis_index("c")
    sid = lax.axis_index("s")
    wid = sid * _NC + cid
    iota2 = 2 * lax.iota(jnp.int32, 16)
    a_bufs = (a_v0, a_v1)
    b_bufs = (b_v0, b_v1)
    o_bufs = (o_v0, o_v1)
    si = (si0, si1)
    so = (so0, so1)

    def a_src(t):
        batch = wid * _NB_W + t // _NCH
        r0 = (t % _NCH) * _NR
        return a_hbm.at[batch, pl.ds(r0 * _H, _NR * _H)]

    def b_src(t):
        batch = wid * _NB_W + t // _NCH
        r0 = (t % _NCH) * _NR
        return b_hbm.at[batch, pl.ds(r0 * _H, _NR * _H)]

    def o_dst(t):
        batch = wid * _NB_W + t // _NCH
        r0 = (t % _NCH) * _NR
        return out_hbm.at[batch, pl.ds(r0, _NR)]

    def issue_in(t, s):
        pltpu.async_copy(a_src(t), a_bufs[s], si[s])
        pltpu.async_copy(b_src(t), b_bufs[s], si[s])

    def wait_in(t, s):
        pltpu.make_async_copy(a_src(t), a_bufs[s], si[s]).wait()
        pltpu.make_async_copy(b_src(t), b_bufs[s], si[s]).wait()

    issue_in(0, 0)
    issue_in(1, 1)

    def interleave_chunk(a_v, b_v, o_v):
        def pair_body(r2, carry2):
            r = 2 * r2
            for p in range(2):
                abase = (r + p) * _H
                row16 = jnp.full((16,), r + p, dtype=jnp.int32)
                for i in range(_H // 16):
                    idx = (p + 32 * i) + iota2
                    plsc.store_scatter(o_v, [row16, idx],
                                       a_v[pl.ds(abase + 16 * i, 16)])
                    idxb = ((1 - p) + 32 * i) + iota2
                    plsc.store_scatter(o_v, [row16, idxb],
                                       b_v[pl.ds(abase + 16 * i, 16)])
            return carry2

        lax.fori_loop(0, _NR // 2, pair_body, 0)

    def step(g, carry):
        for s in range(2):
            t = 2 * g + s
            wait_in(t, s)

            @pl.when(t >= 2)
            def _():
                pltpu.make_async_copy(o_bufs[s], o_dst(t - 2), so[s]).wait()

            interleave_chunk(a_bufs[s], b_bufs[s], o_bufs[s])
            pltpu.async_copy(o_bufs[s], o_dst(t), so[s])

            @pl.when(t + 2 < _NT)
            def _():
                issue_in(t + 2, s)

        return carry

    lax.fori_loop(0, _NT // 2, step, 0)
    pltpu.make_async_copy(o_bufs[0], o_dst(_NT - 2), so[0]).wait()
    pltpu.make_async_copy(o_bufs[1], o_dst(_NT - 1), so[1]).wait()


def kernel(phi_a, phi_b):
    return _sc_restore(phi_a, phi_b)


# speedup vs baseline: 11.2802x; 1.0866x over previous
import functools

import jax
import jax.numpy as jnp
from jax import lax
from jax.experimental import pallas as pl
from jax.experimental.pallas import tpu as pltpu
from jax.experimental.pallas import tpu_sc as plsc

_B = 128
_L1, _L2 = 512, 512
_H = _L2 // 2

_NC, _NS = 2, 16
_NW = _NC * _NS
_NB_W = _B // _NW
_NR = 32
_NCH = _L1 // _NR
_NT = _NB_W * _NCH

_mesh = plsc.VectorSubcoreMesh(core_axis_name="c", subcore_axis_name="s")


@functools.partial(
    pl.kernel,
    out_type=jax.ShapeDtypeStruct((_B, _L1, _L2), jnp.float32),
    mesh=_mesh,
    scratch_types=[
        pltpu.VMEM((_NR * _H,), jnp.float32),
        pltpu.VMEM((_NR * _H,), jnp.float32),
        pltpu.VMEM((_NR * _H,), jnp.float32),
        pltpu.VMEM((_NR * _H,), jnp.float32),
        pltpu.VMEM((_NR, _L2), jnp.float32),
        pltpu.VMEM((_NR, _L2), jnp.float32),
        pltpu.SemaphoreType.DMA,
        pltpu.SemaphoreType.DMA,
        pltpu.SemaphoreType.DMA,
        pltpu.SemaphoreType.DMA,
    ],
    compiler_params=pltpu.CompilerParams(needs_layout_passes=False),
)
def _sc_restore(a_hbm, b_hbm, out_hbm,
                a_v0, a_v1, b_v0, b_v1, o_v0, o_v1,
                si0, si1, so0, so1):
    cid = lax.axis_index("c")
    sid = lax.axis_index("s")
    wid = sid * _NC + cid
    iota2 = 2 * lax.iota(jnp.int32, 16)
    a_bufs = (a_v0, a_v1)
    b_bufs = (b_v0, b_v1)
    o_bufs = (o_v0, o_v1)
    si = (si0, si1)
    so = (so0, so1)

    def a_src(t):
        batch = wid * _NB_W + t // _NCH
        r0 = (t % _NCH) * _NR
        return a_hbm.at[batch, pl.ds(r0 * _H, _NR * _H)]

    def b_src(t):
        batch = wid * _NB_W + t // _NCH
        r0 = (t % _NCH) * _NR
        return b_hbm.at[batch, pl.ds(r0 * _H, _NR * _H)]

    def o_dst(t):
        batch = wid * _NB_W + t // _NCH
        r0 = (t % _NCH) * _NR
        return out_hbm.at[batch, pl.ds(r0, _NR)]

    def issue_in(t, s):
        pltpu.async_copy(a_src(t), a_bufs[s], si[s])
        pltpu.async_copy(b_src(t), b_bufs[s], si[s])

    def wait_in(t, s):
        pltpu.make_async_copy(a_src(t), a_bufs[s], si[s]).wait()
        pltpu.make_async_copy(b_src(t), b_bufs[s], si[s]).wait()

    issue_in(0, 0)
    issue_in(1, 1)

    iota16 = lax.iota(jnp.int32, 16)
    idx_lo = (iota16 >> 1)[:, None]
    idx_hi = (8 + (iota16 >> 1))[:, None]
    even_lane = (iota16 & 1) == 0
    _dn = lax.GatherDimensionNumbers(
        offset_dims=(), collapsed_slice_dims=(0,), start_index_map=(0,))

    def _take16(v, idx):
        return lax.gather(v, idx, _dn, slice_sizes=(1,),
                          mode=lax.GatherScatterMode.PROMISE_IN_BOUNDS)

    def interleave_chunk(a_v, b_v, o_v):
        def pair_body(r2, carry2):
            r = 2 * r2
            for p in range(2):
                abase = (r + p) * _H
                for i in range(_H // 16):
                    av = a_v[pl.ds(abase + 16 * i, 16)]
                    bv = b_v[pl.ds(abase + 16 * i, 16)]
                    ga_lo = _take16(av, idx_lo)
                    gb_lo = _take16(bv, idx_lo)
                    ga_hi = _take16(av, idx_hi)
                    gb_hi = _take16(bv, idx_hi)
                    if p == 0:
                        lo = jnp.where(even_lane, ga_lo, gb_lo)
                        hi = jnp.where(even_lane, ga_hi, gb_hi)
                    else:
                        lo = jnp.where(even_lane, gb_lo, ga_lo)
                        hi = jnp.where(even_lane, gb_hi, ga_hi)
                    o_v[r + p, pl.ds(32 * i, 16)] = lo
                    o_v[r + p, pl.ds(32 * i + 16, 16)] = hi
            return carry2

        lax.fori_loop(0, _NR // 2, pair_body, 0)

    def step(g, carry):
        for s in range(2):
            t = 2 * g + s
            wait_in(t, s)

            @pl.when(t >= 2)
            def _():
                pltpu.make_async_copy(o_bufs[s], o_dst(t - 2), so[s]).wait()

            interleave_chunk(a_bufs[s], b_bufs[s], o_bufs[s])
            pltpu.async_copy(o_bufs[s], o_dst(t), so[s])

            @pl.when(t + 2 < _NT)
            def _():
                issue_in(t + 2, s)

        return carry

    lax.fori_loop(0, _NT // 2, step, 0)
    pltpu.make_async_copy(o_bufs[0], o_dst(_NT - 2), so[0]).wait()
    pltpu.make_async_copy(o_bufs[1], o_dst(_NT - 1), so[1]).wait()


def kernel(phi_a, phi_b):
    return _sc_restore(phi_a, phi_b)


# speedup vs baseline: 11.2930x; 1.0011x over previous
import functools

import jax
import jax.numpy as jnp
from jax import lax
from jax.experimental import pallas as pl
from jax.experimental.pallas import tpu as pltpu
from jax.experimental.pallas import tpu_sc as plsc

_B = 128
_L1, _L2 = 512, 512
_H = _L2 // 2

_NC, _NS = 2, 16
_NW = _NC * _NS
_NB_W = _B // _NW
_NR = 32
_NCH = _L1 // _NR
_NT = _NB_W * _NCH

_mesh = plsc.VectorSubcoreMesh(core_axis_name="c", subcore_axis_name="s")


@functools.partial(
    pl.kernel,
    out_type=jax.ShapeDtypeStruct((_B, _L1, _L2), jnp.float32),
    mesh=_mesh,
    scratch_types=[
        pltpu.VMEM((_NR * _H,), jnp.float32),
        pltpu.VMEM((_NR * _H,), jnp.float32),
        pltpu.VMEM((_NR * _H,), jnp.float32),
        pltpu.VMEM((_NR * _H,), jnp.float32),
        pltpu.VMEM((_NR, _L2), jnp.float32),
        pltpu.VMEM((_NR, _L2), jnp.float32),
        pltpu.SemaphoreType.DMA,
        pltpu.SemaphoreType.DMA,
        pltpu.SemaphoreType.DMA,
        pltpu.SemaphoreType.DMA,
    ],
    compiler_params=pltpu.CompilerParams(needs_layout_passes=False),
)
def _sc_restore(a_hbm, b_hbm, out_hbm,
                a_v0, a_v1, b_v0, b_v1, o_v0, o_v1,
                si0, si1, so0, so1):
    cid = lax.axis_index("c")
    sid = lax.axis_index("s")
    wid = sid * _NC + cid
    a_bufs = (a_v0, a_v1)
    b_bufs = (b_v0, b_v1)
    o_bufs = (o_v0, o_v1)
    si = (si0, si1)
    so = (so0, so1)

    def a_src(t):
        batch = wid * _NB_W + t // _NCH
        r0 = (t % _NCH) * _NR
        return a_hbm.at[batch, pl.ds(r0 * _H, _NR * _H)]

    def b_src(t):
        batch = wid * _NB_W + t // _NCH
        r0 = (t % _NCH) * _NR
        return b_hbm.at[batch, pl.ds(r0 * _H, _NR * _H)]

    def o_dst(t):
        batch = wid * _NB_W + t // _NCH
        r0 = (t % _NCH) * _NR
        return out_hbm.at[batch, pl.ds(r0, _NR)]

    def issue_in(t, s):
        pltpu.async_copy(a_src(t), a_bufs[s], si[s])
        pltpu.async_copy(b_src(t), b_bufs[s], si[s])

    def wait_in(t, s):
        pltpu.make_async_copy(a_src(t), a_bufs[s], si[s]).wait()
        pltpu.make_async_copy(b_src(t), b_bufs[s], si[s]).wait()

    issue_in(0, 0)
    issue_in(1, 1)

    iota16 = lax.iota(jnp.int32, 16)
    idx_lo = (iota16 >> 1)[:, None]
    idx_hi = (8 + (iota16 >> 1))[:, None]
    even_lane = (iota16 & 1) == 0
    _dn = lax.GatherDimensionNumbers(
        offset_dims=(), collapsed_slice_dims=(0,), start_index_map=(0,))

    def _take16(v, idx):
        return lax.gather(v, idx, _dn, slice_sizes=(1,),
                          mode=lax.GatherScatterMode.PROMISE_IN_BOUNDS)

    def interleave_chunk(a_v, b_v, o_v):
        def pair_body(r2, carry2):
            r = 2 * r2
            for p in range(2):
                abase = (r + p) * _H
                for i in range(_H // 16):
                    av = a_v[pl.ds(abase + 16 * i, 16)]
                    bv = b_v[pl.ds(abase + 16 * i, 16)]
                    ga_lo = _take16(av, idx_lo)
                    gb_lo = _take16(bv, idx_lo)
                    ga_hi = _take16(av, idx_hi)
                    gb_hi = _take16(bv, idx_hi)
                    if p == 0:
                        lo = jnp.where(even_lane, ga_lo, gb_lo)
                        hi = jnp.where(even_lane, ga_hi, gb_hi)
                    else:
                        lo = jnp.where(even_lane, gb_lo, ga_lo)
                        hi = jnp.where(even_lane, gb_hi, ga_hi)
                    o_v[r + p, pl.ds(32 * i, 16)] = lo
                    o_v[r + p, pl.ds(32 * i + 16, 16)] = hi
            return carry2

        lax.fori_loop(0, _NR // 2, pair_body, 0)

    def step(g, carry):
        for s in range(2):
            t = 2 * g + s
            wait_in(t, s)

            @pl.when(t >= 2)
            def _():
                pltpu.make_async_copy(o_bufs[s], o_dst(t - 2), so[s]).wait()

            interleave_chunk(a_bufs[s], b_bufs[s], o_bufs[s])
            pltpu.async_copy(o_bufs[s], o_dst(t), so[s])

            @pl.when(t + 2 < _NT)
            def _():
                issue_in(t + 2, s)

        return carry

    lax.fori_loop(0, _NT // 2, step, 0)
    pltpu.make_async_copy(o_bufs[0], o_dst(_NT - 2), so[0]).wait()
    pltpu.make_async_copy(o_bufs[1], o_dst(_NT - 1), so[1]).wait()


def kernel(phi_a, phi_b):
    return _sc_restore(phi_a, phi_b)
